# baseline (speedup 1.0000x reference)
import functools

import jax
import jax.numpy as jnp
from jax.experimental import pallas as pl
from jax.experimental.pallas import tpu as pltpu


def _kc_pass_kernel(scal_ref, x_ref, xt_ref, c_ref, *refs, nb, emit_logits):
    if emit_logits:
        logits_ref, cout_ref, b_ref, ct_ref, acc_ref, w_ref = refs
    else:
        cout_ref, b_ref, ct_ref, acc_ref, w_ref = refs
        logits_ref = None

    j = pl.program_id(0)
    neg_inv_tau = scal_ref[0]

    @pl.when(j == 0)
    def _init():
        ct = jnp.transpose(c_ref[...])
        ct_ref[...] = ct
        b_ref[...] = jnp.sum(ct * ct, axis=0, keepdims=True)
        acc_ref[...] = jnp.zeros_like(acc_ref)
        w_ref[...] = jnp.zeros_like(w_ref)

    x = x_ref[...]
    xc_neg2 = jnp.dot(x * (-2.0), ct_ref[...],
                      preferred_element_type=jnp.float32)
    x2 = jnp.sum(x * x, axis=1, keepdims=True)
    dist = (x2 + xc_neg2) + b_ref[...]
    logits = dist * neg_inv_tau
    if emit_logits:
        logits_ref[...] = logits

    m = jnp.max(logits, axis=1, keepdims=True)
    if emit_logits:
        e = jnp.exp(logits - m)
    else:
        e = jnp.exp2(logits - m)
    s = jnp.sum(e, axis=1, keepdims=True)
    assign = e / s

    w_ref[...] += jnp.sum(assign, axis=0, keepdims=True)
    acc_ref[...] += jnp.dot(xt_ref[...], assign,
                            preferred_element_type=jnp.float32)

    @pl.when(j == nb - 1)
    def _finish():
        cout_ref[...] = jnp.transpose(
            acc_ref[...] / (w_ref[...] + 1e-8))


def _run_pass(scal, x, xt, c, *, block_n, emit_logits, interpret=False):
    n, d = x.shape
    k = c.shape[0]
    nb = n // block_n
    scratch = [
        pltpu.VMEM((1, k), jnp.float32),
        pltpu.VMEM((d, k), jnp.float32),
        pltpu.VMEM((d, k), jnp.float32),
        pltpu.VMEM((1, k), jnp.float32),
    ]
    in_specs = [
        pl.BlockSpec(memory_space=pltpu.SMEM),
        pl.BlockSpec((block_n, d), lambda j: (j, 0)),
        pl.BlockSpec((d, block_n), lambda j: (0, j)),
        pl.BlockSpec((k, d), lambda j: (0, 0)),
    ]
    ct_spec = pl.BlockSpec((k, d), lambda j: (0, 0))
    ct_shape = jax.ShapeDtypeStruct((k, d), jnp.float32)
    if emit_logits:
        out_specs = [pl.BlockSpec((block_n, k), lambda j: (j, 0)), ct_spec]
        out_shape = [jax.ShapeDtypeStruct((n, k), jnp.float32), ct_shape]
    else:
        out_specs = ct_spec
        out_shape = ct_shape
    return pl.pallas_call(
        functools.partial(_kc_pass_kernel, nb=nb, emit_logits=emit_logits),
        grid=(nb,),
        in_specs=in_specs,
        out_specs=out_specs,
        out_shape=out_shape,
        scratch_shapes=scratch,
        interpret=interpret,
    )(scal, x, xt, c)


def kernel(problem_reps, centroids, kmeans_log_tau):
    neg_inv_tau = -1.0 / jnp.exp(kmeans_log_tau)
    log2e = jnp.float32(1.4426950408889634)
    x = problem_reps
    xt = jnp.transpose(x)
    block_n = 4096
    c1 = _run_pass(neg_inv_tau * log2e, x, xt, centroids,
                   block_n=block_n, emit_logits=False)
    logits, c2 = _run_pass(neg_inv_tau, x, xt, c1,
                           block_n=block_n, emit_logits=True)
    return logits, c2


# speedup vs baseline: 1.0414x; 1.0414x over previous
import functools

import jax
import jax.numpy as jnp
from jax.experimental import pallas as pl
from jax.experimental.pallas import tpu as pltpu


def _kc_pass_kernel(scal_ref, x_ref, xt_ref, ct_ref, *refs, nb, emit_logits):
    if emit_logits:
        logits_ref, cout_t_ref, b_ref, acc_ref, w_ref = refs
    else:
        cout_t_ref, b_ref, acc_ref, w_ref = refs
        logits_ref = None

    j = pl.program_id(0)
    neg_inv_tau = scal_ref[0]

    @pl.when(j == 0)
    def _init():
        ct = ct_ref[...]
        b_ref[...] = jnp.sum(ct * ct, axis=0, keepdims=True)
        acc_ref[...] = jnp.zeros_like(acc_ref)
        w_ref[...] = jnp.zeros_like(w_ref)

    x = x_ref[...]
    xc_neg2 = jnp.dot(x * (-2.0), ct_ref[...],
                      preferred_element_type=jnp.float32)
    x2 = jnp.sum(x * x, axis=1, keepdims=True)
    dist = (x2 + xc_neg2) + b_ref[...]
    logits = dist * neg_inv_tau
    if emit_logits:
        logits_ref[...] = logits

    m = jnp.max(logits, axis=1, keepdims=True)
    if emit_logits:
        e = jnp.exp(logits - m)
    else:
        e = jnp.exp2(logits - m)
    s = jnp.sum(e, axis=1, keepdims=True)
    assign = e / s

    w_ref[...] += jnp.sum(assign, axis=0, keepdims=True)
    acc_ref[...] += jnp.dot(xt_ref[...], assign,
                            preferred_element_type=jnp.float32)

    @pl.when(j == nb - 1)
    def _finish():
        cout_t_ref[...] = acc_ref[...] / (w_ref[...] + 1e-8)


def _run_pass(scal, x, xt, ct, *, block_n, emit_logits, interpret=False):
    n, d = x.shape
    k = ct.shape[1]
    nb = n // block_n
    scratch = [
        pltpu.VMEM((1, k), jnp.float32),
        pltpu.VMEM((d, k), jnp.float32),
        pltpu.VMEM((1, k), jnp.float32),
    ]
    in_specs = [
        pl.BlockSpec(memory_space=pltpu.SMEM),
        pl.BlockSpec((block_n, d), lambda j: (j, 0)),
        pl.BlockSpec((d, block_n), lambda j: (0, j)),
        pl.BlockSpec((d, k), lambda j: (0, 0)),
    ]
    ct_spec = pl.BlockSpec((d, k), lambda j: (0, 0))
    ct_shape = jax.ShapeDtypeStruct((d, k), jnp.float32)
    if emit_logits:
        out_specs = [pl.BlockSpec((block_n, k), lambda j: (j, 0)), ct_spec]
        out_shape = [jax.ShapeDtypeStruct((n, k), jnp.float32), ct_shape]
    else:
        out_specs = ct_spec
        out_shape = ct_shape
    return pl.pallas_call(
        functools.partial(_kc_pass_kernel, nb=nb, emit_logits=emit_logits),
        grid=(nb,),
        in_specs=in_specs,
        out_specs=out_specs,
        out_shape=out_shape,
        scratch_shapes=scratch,
        interpret=interpret,
    )(scal, x, xt, ct)


def kernel(problem_reps, centroids, kmeans_log_tau):
    neg_inv_tau = -1.0 / jnp.exp(kmeans_log_tau)
    log2e = jnp.float32(1.4426950408889634)
    x = problem_reps
    xt = jnp.transpose(x)
    ct0 = jnp.transpose(centroids)
    block_n = 4096
    c1t = _run_pass(neg_inv_tau * log2e, x, xt, ct0,
                    block_n=block_n, emit_logits=False)
    logits, c2t = _run_pass(neg_inv_tau, x, xt, c1t,
                            block_n=block_n, emit_logits=True)
    return logits, jnp.transpose(c2t)
